# Initial kernel scaffold; baseline (speedup 1.0000x reference)
import jax
import jax.numpy as jnp
from jax.experimental import pallas as pl


def kernel(x, edge_index, W1, b1, W2, b2):
    raise NotImplementedError("write your pallas kernel here")


# speedup vs baseline: 15.0426x; 15.0426x over previous
import functools

import jax
import jax.numpy as jnp
from jax import lax
from jax.experimental import pallas as pl
from jax.experimental.pallas import tpu as pltpu
from jax.experimental.pallas import tpu_sc as plsc

N = 10000
NP = 10240
D = 128
E = 320000
NC = 2
NS = 16
NW = NC * NS
C = 128
NCHUNK = E // C
BASE_CHUNKS = NCHUNK // NW
EXTRA = NCHUNK - BASE_CHUNKS * NW
RPT = NP // NS
DW = 128

_mesh = plsc.VectorSubcoreMesh(core_axis_name="c", subcore_axis_name="s")


@functools.partial(
    pl.kernel,
    out_type=jax.ShapeDtypeStruct((NC, NP, D), jnp.float32),
    mesh=_mesh,
    scratch_types=[
        pltpu.VMEM((C,), jnp.int32),
        pltpu.VMEM((C,), jnp.int32),
        pltpu.VMEM((C, D), jnp.float32),
        pltpu.VMEM_SHARED((NP, D), jnp.float32),
        pltpu.SemaphoreType.DMA,
    ],
)
def _edge_scatter(xs_hbm, src_hbm, dst_hbm, zeros_hbm, out_hbm,
                  src_v, dst_v, rows_v, acc_sh, sem):
    cid = lax.axis_index("c")
    sid = lax.axis_index("s")
    wid = cid * NS + sid
    roff = pl.multiple_of(sid * RPT, 8)

    pltpu.sync_copy(zeros_hbm.at[pl.ds(roff, RPT)], acc_sh.at[pl.ds(roff, RPT)])
    plsc.subcore_barrier()

    nj = BASE_CHUNKS + jnp.where(wid < EXTRA, 1, 0)

    def body(j, carry):
        base = pl.multiple_of((wid + j * NW) * C, C)
        pltpu.sync_copy(src_hbm.at[pl.ds(base, C)], src_v)
        pltpu.sync_copy(dst_hbm.at[pl.ds(base, C)], dst_v)
        pltpu.async_copy(xs_hbm.at[src_v], rows_v, sem).wait()
        pltpu.sync_copy(rows_v, acc_sh.at[dst_v], add=True)
        return carry

    lax.fori_loop(0, nj, body, 0)
    plsc.subcore_barrier()

    pltpu.sync_copy(acc_sh.at[pl.ds(roff, RPT)],
                    out_hbm.at[cid, pl.ds(roff, RPT)])


@functools.partial(
    pl.kernel,
    out_type=jax.ShapeDtypeStruct((NC, NP, DW), jnp.float32),
    mesh=_mesh,
    scratch_types=[
        pltpu.VMEM((C,), jnp.int32),
        pltpu.VMEM((C, DW), jnp.float32),
        pltpu.VMEM_SHARED((NP, DW), jnp.float32),
    ],
)
def _deg_scatter(dst_hbm, ones_hbm, zeros_hbm, out_hbm,
                 dst_v, ones_v, acc_sh):
    cid = lax.axis_index("c")
    sid = lax.axis_index("s")
    wid = cid * NS + sid
    roff = pl.multiple_of(sid * RPT, 8)

    pltpu.sync_copy(ones_hbm, ones_v)
    pltpu.sync_copy(zeros_hbm.at[pl.ds(roff, RPT)], acc_sh.at[pl.ds(roff, RPT)])
    plsc.subcore_barrier()

    nj = BASE_CHUNKS + jnp.where(wid < EXTRA, 1, 0)

    def body(j, carry):
        base = pl.multiple_of((wid + j * NW) * C, C)
        pltpu.sync_copy(dst_hbm.at[pl.ds(base, C)], dst_v)
        pltpu.sync_copy(ones_v, acc_sh.at[dst_v], add=True)
        return carry

    lax.fori_loop(0, nj, body, 0)
    plsc.subcore_barrier()

    pltpu.sync_copy(acc_sh.at[pl.ds(roff, RPT)],
                    out_hbm.at[cid, pl.ds(roff, RPT)])


B = 2000
_GRID = N // B


def _tc1_body(deg_ref, x_ref, w_ref, dinv_ref, xs_ref):
    dp = deg_ref[...]
    deg = dp[0, :, 0:1] + dp[1, :, 0:1] + 1.0
    dinv = lax.rsqrt(deg)
    xw = jnp.dot(x_ref[...], w_ref[...], preferred_element_type=jnp.float32)
    dinvb = jnp.broadcast_to(dinv, (B, D))
    dinv_ref[...] = dinvb
    xs_ref[...] = xw * dinvb


_tc1 = pl.pallas_call(
    _tc1_body,
    grid=(_GRID,),
    in_specs=[
        pl.BlockSpec((NC, B, DW), lambda i: (0, i, 0)),
        pl.BlockSpec((B, D), lambda i: (i, 0)),
        pl.BlockSpec((D, D), lambda i: (0, 0)),
    ],
    out_specs=[
        pl.BlockSpec((B, D), lambda i: (i, 0)),
        pl.BlockSpec((B, D), lambda i: (i, 0)),
    ],
    out_shape=[
        jax.ShapeDtypeStruct((N, D), jnp.float32),
        jax.ShapeDtypeStruct((N, D), jnp.float32),
    ],
)


def _tc2_body(p_ref, xs1_ref, dinv_ref, b1_ref, w2_ref, xs2_ref):
    pp = p_ref[...]
    s = pp[0] + pp[1] + xs1_ref[...]
    h = jnp.maximum(dinv_ref[...] * s + b1_ref[...], 0.0)
    hw = jnp.dot(h, w2_ref[...], preferred_element_type=jnp.float32)
    xs2_ref[...] = hw * dinv_ref[...]


_tc2 = pl.pallas_call(
    _tc2_body,
    grid=(_GRID,),
    in_specs=[
        pl.BlockSpec((NC, B, D), lambda i: (0, i, 0)),
        pl.BlockSpec((B, D), lambda i: (i, 0)),
        pl.BlockSpec((B, D), lambda i: (i, 0)),
        pl.BlockSpec((1, D), lambda i: (0, 0)),
        pl.BlockSpec((D, D), lambda i: (0, 0)),
    ],
    out_specs=pl.BlockSpec((B, D), lambda i: (i, 0)),
    out_shape=jax.ShapeDtypeStruct((N, D), jnp.float32),
)


def _tc3_body(q_ref, xs2_ref, dinv_ref, b2_ref, out_ref):
    qq = q_ref[...]
    s = qq[0] + qq[1] + xs2_ref[...]
    out_ref[...] = dinv_ref[...] * s + b2_ref[...]


_tc3 = pl.pallas_call(
    _tc3_body,
    grid=(_GRID,),
    in_specs=[
        pl.BlockSpec((NC, B, D), lambda i: (0, i, 0)),
        pl.BlockSpec((B, D), lambda i: (i, 0)),
        pl.BlockSpec((B, D), lambda i: (i, 0)),
        pl.BlockSpec((1, D), lambda i: (0, 0)),
    ],
    out_specs=pl.BlockSpec((B, D), lambda i: (i, 0)),
    out_shape=jax.ShapeDtypeStruct((N, D), jnp.float32),
)


def kernel(x, edge_index, W1, b1, W2, b2):
    ei = edge_index.astype(jnp.int32)
    src = ei[0]
    dst = ei[1]
    zeros = jnp.zeros((NP, D), jnp.float32)
    ones = jnp.ones((C, DW), jnp.float32)

    degp = _deg_scatter(dst, ones, zeros)
    dinv, xs1 = _tc1(degp, x, W1)
    p = _edge_scatter(xs1, src, dst, zeros)
    xs2 = _tc2(p, xs1, dinv, b1.reshape(1, D), W2)
    q = _edge_scatter(xs2, src, dst, zeros)
    out = _tc3(q, xs2, dinv, b2.reshape(1, D))
    return out
